# baseline (speedup 1.0000x reference)
import functools

import jax
import jax.numpy as jnp
from jax import lax
from jax.experimental import pallas as pl
from jax.experimental.pallas import tpu as pltpu
from jax.experimental.pallas import tpu_sc as plsc

N = 10000
D = 64
E = 160000

NC = 2
NS = 16
NW = NC * NS
EPT = E // NW
CH = 128
NCHT = E // CH
CPT = 40
NCHP = NW * CPT
NP = 10240
RPT = NP // NS
HI = jax.lax.Precision.HIGHEST

_mesh = lambda: plsc.VectorSubcoreMesh(core_axis_name="c", subcore_axis_name="s")


def _degrees(f_heter, f_h0, f_h1, f_hm0, f_hm1):
    @functools.partial(
        pl.kernel,
        out_type=jax.ShapeDtypeStruct((NW * 9 * N,), jnp.float32),
        mesh=_mesh(),
        compiler_params=pltpu.CompilerParams(needs_layout_passes=False),
        scratch_types=[
            pltpu.VMEM((9 * N,), jnp.float32),
            pltpu.VMEM((EPT + 16,), jnp.int32),
        ],
    )
    def k(heter_h, h0_h, h1_h, hm0_h, hm1_h, out_h, hist_v, idx_v):
        cid = lax.axis_index("c")
        sid = lax.axis_index("s")
        wid = cid * NS + sid
        zeros16 = jnp.zeros((16,), jnp.float32)

        def zb(i, _):
            hist_v[pl.ds(i * 16, 16)] = zeros16
            return 0

        lax.fori_loop(0, (9 * N) // 16, zb, 0)

        base_e = wid * EPT
        ones = jnp.ones((16,), jnp.float32)
        tail_mask = lax.iota(jnp.int32, 16) < (EPT - (EPT // 16) * 16)
        jobs = [
            (heter_h, 0, 0),
            (h0_h, 0, 1), (h0_h, 1, 2),
            (h1_h, 0, 3), (h1_h, 1, 4),
            (hm0_h, 0, 5), (hm0_h, 1, 6),
            (hm1_h, 0, 7), (hm1_h, 1, 8),
        ]
        for src, row, j in jobs:
            pltpu.sync_copy(src.at[pl.ds(row * E + base_e, EPT)],
                            idx_v.at[pl.ds(0, EPT)])
            off = jnp.int32(j * N)

            def body(i, _):
                ix = idx_v[pl.ds(i * 16, 16)] + off
                plsc.addupdate_scatter(hist_v, [ix], ones)
                return 0

            lax.fori_loop(0, EPT // 16, body, 0)
            ixt = idx_v[pl.ds((EPT // 16) * 16, 16)] + off
            plsc.addupdate_scatter(hist_v, [ixt], ones, mask=tail_mask)

        pltpu.sync_copy(hist_v, out_h.at[pl.ds(wid * 9 * N, 9 * N)])

    return k(f_heter, f_h0, f_h1, f_hm0, f_hm1)


def _spmm(z, src2, dst2, zeros_stripe):
    @functools.partial(
        pl.kernel,
        out_type=jax.ShapeDtypeStruct((2 * NP, D), jnp.float32),
        mesh=_mesh(),
        compiler_params=pltpu.CompilerParams(needs_layout_passes=False,
                                             use_tc_tiling_on_sc=False),
        scratch_types=[
            pltpu.VMEM_SHARED((NP, D), jnp.float32),
            pltpu.VMEM_SHARED((NP, D), jnp.float32),
            pltpu.VMEM((CH, D), jnp.float32),
            pltpu.VMEM((CH, D), jnp.float32),
            pltpu.VMEM((CPT, CH), jnp.int32),
            pltpu.VMEM((CPT, CH), jnp.int32),
            pltpu.SemaphoreType.DMA,
            pltpu.SemaphoreType.DMA,
            pltpu.SemaphoreType.DMA,
            pltpu.SemaphoreType.DMA,
        ],
    )
    def k(z_h, src_h, dst_h, zz_h, out_h,
          acc_s, z_s, rows_a, rows_b, si_big, gi_big, sg_a, sg_b, ss_a, ss_b):
        cid = lax.axis_index("c")
        sid = lax.axis_index("s")
        wid = cid * NS + sid
        row0 = sid * RPT

        pltpu.sync_copy(zz_h, acc_s.at[pl.ds(row0, RPT)])
        pltpu.sync_copy(z_h.at[pl.ds(row0, RPT)], z_s.at[pl.ds(row0, RPT)])

        r0 = wid * CPT
        pltpu.sync_copy(dst_h.at[pl.ds(r0, CPT)], gi_big)
        pltpu.sync_copy(src_h.at[pl.ds(r0, CPT)], si_big)
        nc = jnp.where(wid == NW - 1, NCHT - (NW - 1) * CPT, CPT)

        plsc.subcore_barrier()

        pltpu.async_copy(z_s.at[gi_big.at[0]], rows_a, sg_a)

        def body(kk, _):
            a1 = 2 * kk
            b1 = 2 * kk + 1
            a2 = 2 * kk + 2
            pltpu.async_copy(z_s.at[gi_big.at[b1]], rows_b, sg_b)
            pltpu.make_async_copy(z_s.at[gi_big.at[a1]], rows_a, sg_a).wait()
            pltpu.async_copy(rows_a, acc_s.at[si_big.at[a1]], ss_a, add=True)
            pltpu.make_async_copy(rows_a, acc_s.at[si_big.at[a1]], ss_a).wait()
            pltpu.async_copy(z_s.at[gi_big.at[a2]], rows_a, sg_a)
            pltpu.make_async_copy(z_s.at[gi_big.at[b1]], rows_b, sg_b).wait()
            pltpu.async_copy(rows_b, acc_s.at[si_big.at[b1]], ss_b, add=True)
            pltpu.make_async_copy(rows_b, acc_s.at[si_big.at[b1]], ss_b).wait()
            return 0

        lax.fori_loop(0, (nc - 2) // 2, body, 0)

        pltpu.make_async_copy(z_s.at[gi_big.at[0]], rows_a, sg_a).wait()
        pltpu.sync_copy(rows_a, acc_s.at[si_big.at[nc - 2]], add=True)
        pltpu.async_copy(z_s.at[gi_big.at[nc - 1]], rows_b, sg_b).wait()
        pltpu.sync_copy(rows_b, acc_s.at[si_big.at[nc - 1]], add=True)

        plsc.subcore_barrier()
        pltpu.sync_copy(acc_s.at[pl.ds(row0, RPT)],
                        out_h.at[pl.ds(cid * NP + row0, RPT)])

    return k(z, src2, dst2, zeros_stripe)


_R = 1000
_G = N // _R
_RP = 1280
_GP = NP // _RP


def _row_spec(shape_tail):
    return pl.BlockSpec((_R,) + shape_tail, lambda i: (i,) + (0,) * len(shape_tail))


def _rowp_spec(shape_tail):
    return pl.BlockSpec((_RP,) + shape_tail, lambda i: (i,) + (0,) * len(shape_tail))


def _full_spec(shape):
    return pl.BlockSpec(shape, lambda i: (0,) * len(shape))


def _prep(emb, W0, b0, W1, b1, theta):
    def body(e_r, w0_r, b0_r, w1_r, b1_r, th_r, u0_r, c_r, xt_r):
        e = e_r[...]
        u0 = e * jax.nn.sigmoid(jnp.dot(e, w0_r[...], precision=HI) + b0_r[...])
        c = e * jax.nn.sigmoid(jnp.dot(e, w1_r[...], precision=HI) + b1_r[...])
        u0_r[...] = u0
        c_r[...] = c
        xt_r[...] = jnp.dot(c, th_r[...], precision=HI)

    o = jax.ShapeDtypeStruct((NP, D), jnp.float32)
    return pl.pallas_call(
        body,
        grid=(_GP,),
        in_specs=[_rowp_spec((D,)), _full_spec((D, D)), _full_spec((1, D)),
                  _full_spec((D, D)), _full_spec((1, D)), _full_spec((D, D))],
        out_specs=[_rowp_spec((D,))] * 3,
        out_shape=[o, o, o],
    )(emb, W0, b0, W1, b1, theta)


def _factors(per_tile):
    def body(h_r, out_r):
        c = jnp.sum(h_r[...], axis=0)
        deg = c[0:1]
        dsi = lax.rsqrt(jnp.where(deg == 0.0, 1.0, deg))
        inv = jnp.where(c[1:9] > 0.0, 1.0 / jnp.where(c[1:9] > 0.0, c[1:9], 1.0), 0.0)
        out_r[...] = jnp.concatenate([dsi, inv], axis=0)

    return pl.pallas_call(
        body,
        grid=(1,),
        in_specs=[_full_spec((NW, 9, N))],
        out_specs=_full_spec((9, N)),
        out_shape=jax.ShapeDtypeStruct((9, N), jnp.float32),
    )(per_tile)


def _norm_scale(x, dsi):
    def body(x_r, d_r, z_r):
        x = x_r[...]
        nrm = jnp.sqrt(jnp.sum(x * x, axis=1, keepdims=True)) + 1e-12
        z_r[...] = x * d_r[...] / nrm

    return pl.pallas_call(
        body,
        grid=(_GP,),
        in_specs=[_rowp_spec((D,)), _rowp_spec((1,))],
        out_specs=_rowp_spec((D,)),
        out_shape=jax.ShapeDtypeStruct((NP, D), jnp.float32),
    )(x, dsi)


def _combine(p, dsi, S):
    def body(p0_r, p1_r, d_r, s_r, z_r, so_r):
        d = d_r[...]
        x = (p0_r[...] + p1_r[...]) * d
        so_r[...] = s_r[...] + x
        nrm = jnp.sqrt(jnp.sum(x * x, axis=1, keepdims=True)) + 1e-12
        z_r[...] = x * d / nrm

    o = jax.ShapeDtypeStruct((NP, D), jnp.float32)
    return pl.pallas_call(
        body,
        grid=(_GP,),
        in_specs=[_rowp_spec((D,))] * 2 + [_rowp_spec((1,)), _rowp_spec((D,))],
        out_specs=[_rowp_spec((D,))] * 2,
        out_shape=[o, o],
    )(p[:NP], p[NP:], dsi, S)


def _scale2(p, binv):
    def body(p0_r, p1_r, b_r, m_r):
        m_r[...] = (p0_r[...] + p1_r[...]) * b_r[...]

    return pl.pallas_call(
        body,
        grid=(_GP,),
        in_specs=[_rowp_spec((D,))] * 2 + [_rowp_spec((1,))],
        out_specs=_rowp_spec((D,)),
        out_shape=jax.ShapeDtypeStruct((NP, D), jnp.float32),
    )(p[:NP], p[NP:], binv)


def _gelu(x):
    return 0.5 * x * (1.0 + lax.erf(x * (2.0 ** -0.5)))


def _fusion(q0, dinv0, q1, dinv1, hg_bias, fW1, fb1, fW2, fb2):
    def body(q00_r, q01_r, d0_r, q10_r, q11_r, d1_r, bias_r,
             w1_r, b1_r, w2_r, out_r):
        bias = bias_r[...]
        h0 = (q00_r[...] + q01_r[...]) * d0_r[...] + bias
        h1 = (q10_r[...] + q11_r[...]) * d1_r[...] + bias
        w1t = w1_r[...]
        dn = (((1,), (1,)), ((), ()))
        g0 = _gelu(lax.dot_general(h0, w1t, dn, precision=HI) + b1_r[...])
        g1 = _gelu(lax.dot_general(h1, w1t, dn, precision=HI) + b1_r[...])
        w2 = w2_r[...]
        s0 = lax.dot_general(g0, w2, dn, precision=HI)
        s1 = lax.dot_general(g1, w2, dn, precision=HI)
        mx = jnp.maximum(s0, s1)
        e0 = jnp.exp(s0 - mx)
        e1 = jnp.exp(s1 - mx)
        out_r[...] = (e0 * h0 + e1 * h1) / (e0 + e1)

    return pl.pallas_call(
        body,
        grid=(_G,),
        in_specs=[_row_spec((D,)), _row_spec((D,)), _row_spec((1,)),
                  _row_spec((D,)), _row_spec((D,)), _row_spec((1,)),
                  _full_spec((1, D)),
                  _full_spec((D, D)), _full_spec((1, D)),
                  _full_spec((1, D))],
        out_specs=_row_spec((D,)),
        out_shape=jax.ShapeDtypeStruct((N, D), jnp.float32),
    )(q0[:N], q0[NP:NP + N], dinv0, q1[:N], q1[NP:NP + N], dinv1,
      hg_bias.reshape(1, D), fW1, fb1.reshape(1, D), fW2)


def _attention(u0, S, h_emb, hm_emb, att, att_m):
    def body(u0_r, s_r, h_r, hm_r, att_r, am_r, out_r):
        u = 0.1 * u0_r[...] + 0.15 * s_r[...]
        h = h_r[...]
        hm = hm_r[...]
        dn = (((1,), (1,)), ((), ()))
        v = lax.dot_general(am_r[...], att_r[...], dn, precision=HI)
        dn2 = (((1,), (0,)), ((), ()))
        wu = lax.dot_general(u, v, dn2, precision=HI)
        wh = lax.dot_general(h, v, dn2, precision=HI)
        wm = lax.dot_general(hm, v, dn2, precision=HI)
        mx = jnp.maximum(jnp.maximum(wu, wh), wm)
        eu = jnp.exp(wu - mx)
        eh = jnp.exp(wh - mx)
        em = jnp.exp(wm - mx)
        out_r[...] = (eu * u + eh * h + em * hm) / (eu + eh + em)

    return pl.pallas_call(
        body,
        grid=(_G,),
        in_specs=[_row_spec((D,))] * 4 + [_full_spec((1, D)), _full_spec((D, D))],
        out_specs=_row_spec((D,)),
        out_shape=jax.ShapeDtypeStruct((N, D), jnp.float32),
    )(u0, S, h_emb, hm_emb, att, att_m)


def kernel(heter_edge_index, hyper_edge_index_0, hyper_edge_index_1,
           hyper_micro_edge_index_0, hyper_micro_edge_index_1, emb_table,
           W0, b0, W1, b1, theta, hg_bias, att, att_m,
           fus_W1, fus_b1, fus_W2, fus_b2):
    f_heter = heter_edge_index.reshape(-1)
    f_h0 = hyper_edge_index_0.reshape(-1)
    f_h1 = hyper_edge_index_1.reshape(-1)
    f_hm0 = hyper_micro_edge_index_0.reshape(-1)
    f_hm1 = hyper_micro_edge_index_1.reshape(-1)

    hist = _degrees(f_heter, f_h0, f_h1, f_hm0, f_hm1)
    fac = _factors(hist.reshape(NW, 9, N))
    padc = jnp.zeros((NP - N, 1), jnp.float32)

    def _padded(row):
        return jnp.concatenate([row.reshape(N, 1), padc], axis=0)

    dsi = _padded(fac[0])
    dinv = [fac[i].reshape(N, 1) for i in (1, 3, 5, 7)]
    binv = [_padded(fac[i]) for i in (2, 4, 6, 8)]

    emb_p = jnp.concatenate([emb_table, jnp.zeros((NP - N, D), jnp.float32)], axis=0)
    u0, c_emb, xt = _prep(emb_p, W0, b0, W1, b1, theta)
    del c_emb

    zz = jnp.zeros((RPT, D), jnp.float32)

    def _chunked(ix):
        pad = jnp.zeros((NCHP - NCHT, CH), jnp.int32)
        return jnp.concatenate([ix.reshape(NCHT, CH), pad], axis=0)

    z = _norm_scale(u0, dsi)
    S = jnp.zeros((NP, D), jnp.float32)
    src = _chunked(heter_edge_index[0])
    dst = _chunked(heter_edge_index[1])
    for _ in range(6):
        p = _spmm(z, src, dst, zz)
        z, S = _combine(p, dsi, S)

    outs = []
    for i, ei in enumerate([hyper_edge_index_0, hyper_edge_index_1,
                            hyper_micro_edge_index_0, hyper_micro_edge_index_1]):
        node, he = _chunked(ei[0]), _chunked(ei[1])
        pm = _spmm(xt, he, node, zz)
        m = _scale2(pm, binv[i])
        outs.append(_spmm(m, node, he, zz))

    h_emb = _fusion(outs[0], dinv[0], outs[1], dinv[1],
                    hg_bias, fus_W1, fus_b1, fus_W2, fus_b2)
    h_emb_micro = _fusion(outs[2], dinv[2], outs[3], dinv[3],
                          hg_bias, fus_W1, fus_b1, fus_W2, fus_b2)

    return _attention(u0[:N], S[:N], h_emb, h_emb_micro, att, att_m)


# speedup vs baseline: 1.0873x; 1.0873x over previous
import functools

import jax
import jax.numpy as jnp
from jax import lax
from jax.experimental import pallas as pl
from jax.experimental.pallas import tpu as pltpu
from jax.experimental.pallas import tpu_sc as plsc

N = 10000
D = 64
E = 160000

NC = 2
NS = 16
NW = NC * NS
EPT = E // NW
CH = 128
NCHT = E // CH
CPT = 40
NCHP = NW * CPT
NP = 10240
RPT = NP // NS
HI = jax.lax.Precision.HIGHEST

_mesh = lambda: plsc.VectorSubcoreMesh(core_axis_name="c", subcore_axis_name="s")


def _degrees(f_heter, f_h0, f_h1, f_hm0, f_hm1):
    @functools.partial(
        pl.kernel,
        out_type=jax.ShapeDtypeStruct((NW * 9 * N,), jnp.float32),
        mesh=_mesh(),
        compiler_params=pltpu.CompilerParams(needs_layout_passes=False),
        scratch_types=[
            pltpu.VMEM((9 * N,), jnp.float32),
            pltpu.VMEM((EPT + 16,), jnp.int32),
        ],
    )
    def k(heter_h, h0_h, h1_h, hm0_h, hm1_h, out_h, hist_v, idx_v):
        cid = lax.axis_index("c")
        sid = lax.axis_index("s")
        wid = cid * NS + sid
        zeros16 = jnp.zeros((16,), jnp.float32)

        def zb(i, _):
            hist_v[pl.ds(i * 16, 16)] = zeros16
            return 0

        lax.fori_loop(0, (9 * N) // 16, zb, 0)

        base_e = wid * EPT
        ones = jnp.ones((16,), jnp.float32)
        tail_mask = lax.iota(jnp.int32, 16) < (EPT - (EPT // 16) * 16)
        jobs = [
            (heter_h, 0, 0),
            (h0_h, 0, 1), (h0_h, 1, 2),
            (h1_h, 0, 3), (h1_h, 1, 4),
            (hm0_h, 0, 5), (hm0_h, 1, 6),
            (hm1_h, 0, 7), (hm1_h, 1, 8),
        ]
        for src, row, j in jobs:
            pltpu.sync_copy(src.at[pl.ds(row * E + base_e, EPT)],
                            idx_v.at[pl.ds(0, EPT)])
            off = jnp.int32(j * N)

            def body(i, _):
                ix = idx_v[pl.ds(i * 16, 16)] + off
                plsc.addupdate_scatter(hist_v, [ix], ones)
                return 0

            lax.fori_loop(0, EPT // 16, body, 0)
            ixt = idx_v[pl.ds((EPT // 16) * 16, 16)] + off
            plsc.addupdate_scatter(hist_v, [ixt], ones, mask=tail_mask)

        pltpu.sync_copy(hist_v, out_h.at[pl.ds(wid * 9 * N, 9 * N)])

    return k(f_heter, f_h0, f_h1, f_hm0, f_hm1)


def _spmm(z, src2, dst2, zeros_stripe):
    @functools.partial(
        pl.kernel,
        out_type=jax.ShapeDtypeStruct((2 * NP, D), jnp.float32),
        mesh=_mesh(),
        compiler_params=pltpu.CompilerParams(needs_layout_passes=False,
                                             use_tc_tiling_on_sc=False),
        scratch_types=[
            pltpu.VMEM_SHARED((NP, D), jnp.float32),
            [pltpu.VMEM((CH, D), jnp.float32)] * 4,
            pltpu.VMEM((CPT, CH), jnp.int32),
            pltpu.VMEM((CPT, CH), jnp.int32),
            [pltpu.SemaphoreType.DMA] * 4,
            [pltpu.SemaphoreType.DMA] * 4,
        ],
    )
    def k(z_h, src_h, dst_h, zz_h, out_h,
          acc_s, rows, si_big, gi_big, sg, ss):
        cid = lax.axis_index("c")
        sid = lax.axis_index("s")
        wid = cid * NS + sid
        row0 = sid * RPT

        pltpu.sync_copy(zz_h, acc_s.at[pl.ds(row0, RPT)])

        r0 = wid * CPT
        pltpu.sync_copy(dst_h.at[pl.ds(r0, CPT)], gi_big)
        pltpu.sync_copy(src_h.at[pl.ds(r0, CPT)], si_big)
        nc = jnp.where(wid == NW - 1, NCHT - (NW - 1) * CPT, CPT)

        plsc.subcore_barrier()

        def start_gather(j, b):
            pltpu.async_copy(z_h.at[gi_big.at[j]], rows[b], sg[b])

        def wait_gather(b):
            pltpu.make_async_copy(z_h.at[gi_big.at[0]], rows[b], sg[b]).wait()

        def start_scatter(j, b):
            pltpu.async_copy(rows[b], acc_s.at[si_big.at[j]], ss[b], add=True)

        def wait_scatter(b):
            pltpu.make_async_copy(rows[b], acc_s.at[si_big.at[0]], ss[b]).wait()

        for b in range(4):
            start_gather(b, b)

        def quad(q, _):
            j0 = 4 * q
            for b in range(4):
                wait_gather(b)
                start_scatter(j0 + b, b)
            for b in range(4):
                wait_scatter(b)
                start_gather(j0 + 4 + b, b)
            return 0

        lax.fori_loop(0, (nc - 4) // 4, quad, 0)

        jl = 4 * ((nc - 4) // 4)
        for b in range(4):
            wait_gather(b)
            start_scatter(jl + b, b)
        for b in range(4):
            wait_scatter(b)

        def seq(r, _):
            j = jl + 4 + r
            pltpu.async_copy(z_h.at[gi_big.at[j]], rows[0], sg[0])
            wait_gather(0)
            pltpu.sync_copy(rows[0], acc_s.at[si_big.at[j]], add=True)
            return 0

        lax.fori_loop(0, nc - jl - 4, seq, 0)

        plsc.subcore_barrier()
        pltpu.sync_copy(acc_s.at[pl.ds(row0, RPT)],
                        out_h.at[pl.ds(cid * NP + row0, RPT)])

    return k(z, src2, dst2, zeros_stripe)


_R = 1000
_G = N // _R
_RP = 1280
_GP = NP // _RP


def _row_spec(shape_tail):
    return pl.BlockSpec((_R,) + shape_tail, lambda i: (i,) + (0,) * len(shape_tail))


def _rowp_spec(shape_tail):
    return pl.BlockSpec((_RP,) + shape_tail, lambda i: (i,) + (0,) * len(shape_tail))


def _full_spec(shape):
    return pl.BlockSpec(shape, lambda i: (0,) * len(shape))


def _prep(emb, W0, b0, W1, b1, theta):
    def body(e_r, w0_r, b0_r, w1_r, b1_r, th_r, u0_r, c_r, xt_r):
        e = e_r[...]
        u0 = e * jax.nn.sigmoid(jnp.dot(e, w0_r[...], precision=HI) + b0_r[...])
        c = e * jax.nn.sigmoid(jnp.dot(e, w1_r[...], precision=HI) + b1_r[...])
        u0_r[...] = u0
        c_r[...] = c
        xt_r[...] = jnp.dot(c, th_r[...], precision=HI)

    o = jax.ShapeDtypeStruct((NP, D), jnp.float32)
    return pl.pallas_call(
        body,
        grid=(_GP,),
        in_specs=[_rowp_spec((D,)), _full_spec((D, D)), _full_spec((1, D)),
                  _full_spec((D, D)), _full_spec((1, D)), _full_spec((D, D))],
        out_specs=[_rowp_spec((D,))] * 3,
        out_shape=[o, o, o],
    )(emb, W0, b0, W1, b1, theta)


def _factors(per_tile):
    def body(h_r, out_r):
        c = jnp.sum(h_r[...], axis=0)
        deg = c[0:1]
        dsi = lax.rsqrt(jnp.where(deg == 0.0, 1.0, deg))
        inv = jnp.where(c[1:9] > 0.0, 1.0 / jnp.where(c[1:9] > 0.0, c[1:9], 1.0), 0.0)
        out_r[...] = jnp.concatenate([dsi, inv], axis=0)

    return pl.pallas_call(
        body,
        grid=(1,),
        in_specs=[_full_spec((NW, 9, N))],
        out_specs=_full_spec((9, N)),
        out_shape=jax.ShapeDtypeStruct((9, N), jnp.float32),
    )(per_tile)


def _norm_scale(x, dsi):
    def body(x_r, d_r, z_r):
        x = x_r[...]
        nrm = jnp.sqrt(jnp.sum(x * x, axis=1, keepdims=True)) + 1e-12
        z_r[...] = x * d_r[...] / nrm

    return pl.pallas_call(
        body,
        grid=(_GP,),
        in_specs=[_rowp_spec((D,)), _rowp_spec((1,))],
        out_specs=_rowp_spec((D,)),
        out_shape=jax.ShapeDtypeStruct((NP, D), jnp.float32),
    )(x, dsi)


def _combine(p, dsi, S):
    def body(p0_r, p1_r, d_r, s_r, z_r, so_r):
        d = d_r[...]
        x = (p0_r[...] + p1_r[...]) * d
        so_r[...] = s_r[...] + x
        nrm = jnp.sqrt(jnp.sum(x * x, axis=1, keepdims=True)) + 1e-12
        z_r[...] = x * d / nrm

    o = jax.ShapeDtypeStruct((NP, D), jnp.float32)
    return pl.pallas_call(
        body,
        grid=(_GP,),
        in_specs=[_rowp_spec((D,))] * 2 + [_rowp_spec((1,)), _rowp_spec((D,))],
        out_specs=[_rowp_spec((D,))] * 2,
        out_shape=[o, o],
    )(p[:NP], p[NP:], dsi, S)


def _scale2(p, binv):
    def body(p0_r, p1_r, b_r, m_r):
        m_r[...] = (p0_r[...] + p1_r[...]) * b_r[...]

    return pl.pallas_call(
        body,
        grid=(_GP,),
        in_specs=[_rowp_spec((D,))] * 2 + [_rowp_spec((1,))],
        out_specs=_rowp_spec((D,)),
        out_shape=jax.ShapeDtypeStruct((NP, D), jnp.float32),
    )(p[:NP], p[NP:], binv)


def _gelu(x):
    return 0.5 * x * (1.0 + lax.erf(x * (2.0 ** -0.5)))


def _fusion(q0, dinv0, q1, dinv1, hg_bias, fW1, fb1, fW2, fb2):
    def body(q00_r, q01_r, d0_r, q10_r, q11_r, d1_r, bias_r,
             w1_r, b1_r, w2_r, out_r):
        bias = bias_r[...]
        h0 = (q00_r[...] + q01_r[...]) * d0_r[...] + bias
        h1 = (q10_r[...] + q11_r[...]) * d1_r[...] + bias
        w1t = w1_r[...]
        dn = (((1,), (1,)), ((), ()))
        g0 = _gelu(lax.dot_general(h0, w1t, dn, precision=HI) + b1_r[...])
        g1 = _gelu(lax.dot_general(h1, w1t, dn, precision=HI) + b1_r[...])
        w2 = w2_r[...]
        s0 = lax.dot_general(g0, w2, dn, precision=HI)
        s1 = lax.dot_general(g1, w2, dn, precision=HI)
        mx = jnp.maximum(s0, s1)
        e0 = jnp.exp(s0 - mx)
        e1 = jnp.exp(s1 - mx)
        out_r[...] = (e0 * h0 + e1 * h1) / (e0 + e1)

    return pl.pallas_call(
        body,
        grid=(_G,),
        in_specs=[_row_spec((D,)), _row_spec((D,)), _row_spec((1,)),
                  _row_spec((D,)), _row_spec((D,)), _row_spec((1,)),
                  _full_spec((1, D)),
                  _full_spec((D, D)), _full_spec((1, D)),
                  _full_spec((1, D))],
        out_specs=_row_spec((D,)),
        out_shape=jax.ShapeDtypeStruct((N, D), jnp.float32),
    )(q0[:N], q0[NP:NP + N], dinv0, q1[:N], q1[NP:NP + N], dinv1,
      hg_bias.reshape(1, D), fW1, fb1.reshape(1, D), fW2)


def _attention(u0, S, h_emb, hm_emb, att, att_m):
    def body(u0_r, s_r, h_r, hm_r, att_r, am_r, out_r):
        u = 0.1 * u0_r[...] + 0.15 * s_r[...]
        h = h_r[...]
        hm = hm_r[...]
        dn = (((1,), (1,)), ((), ()))
        v = lax.dot_general(am_r[...], att_r[...], dn, precision=HI)
        dn2 = (((1,), (0,)), ((), ()))
        wu = lax.dot_general(u, v, dn2, precision=HI)
        wh = lax.dot_general(h, v, dn2, precision=HI)
        wm = lax.dot_general(hm, v, dn2, precision=HI)
        mx = jnp.maximum(jnp.maximum(wu, wh), wm)
        eu = jnp.exp(wu - mx)
        eh = jnp.exp(wh - mx)
        em = jnp.exp(wm - mx)
        out_r[...] = (eu * u + eh * h + em * hm) / (eu + eh + em)

    return pl.pallas_call(
        body,
        grid=(_G,),
        in_specs=[_row_spec((D,))] * 4 + [_full_spec((1, D)), _full_spec((D, D))],
        out_specs=_row_spec((D,)),
        out_shape=jax.ShapeDtypeStruct((N, D), jnp.float32),
    )(u0, S, h_emb, hm_emb, att, att_m)


def kernel(heter_edge_index, hyper_edge_index_0, hyper_edge_index_1,
           hyper_micro_edge_index_0, hyper_micro_edge_index_1, emb_table,
           W0, b0, W1, b1, theta, hg_bias, att, att_m,
           fus_W1, fus_b1, fus_W2, fus_b2):
    f_heter = heter_edge_index.reshape(-1)
    f_h0 = hyper_edge_index_0.reshape(-1)
    f_h1 = hyper_edge_index_1.reshape(-1)
    f_hm0 = hyper_micro_edge_index_0.reshape(-1)
    f_hm1 = hyper_micro_edge_index_1.reshape(-1)

    hist = _degrees(f_heter, f_h0, f_h1, f_hm0, f_hm1)
    fac = _factors(hist.reshape(NW, 9, N))
    padc = jnp.zeros((NP - N, 1), jnp.float32)

    def _padded(row):
        return jnp.concatenate([row.reshape(N, 1), padc], axis=0)

    dsi = _padded(fac[0])
    dinv = [fac[i].reshape(N, 1) for i in (1, 3, 5, 7)]
    binv = [_padded(fac[i]) for i in (2, 4, 6, 8)]

    emb_p = jnp.concatenate([emb_table, jnp.zeros((NP - N, D), jnp.float32)], axis=0)
    u0, c_emb, xt = _prep(emb_p, W0, b0, W1, b1, theta)
    del c_emb

    zz = jnp.zeros((RPT, D), jnp.float32)

    def _chunked(ix):
        pad = jnp.zeros((NCHP - NCHT, CH), jnp.int32)
        return jnp.concatenate([ix.reshape(NCHT, CH), pad], axis=0)

    z = _norm_scale(u0, dsi)
    S = jnp.zeros((NP, D), jnp.float32)
    src = _chunked(heter_edge_index[0])
    dst = _chunked(heter_edge_index[1])
    for _ in range(6):
        p = _spmm(z, src, dst, zz)
        z, S = _combine(p, dsi, S)

    outs = []
    for i, ei in enumerate([hyper_edge_index_0, hyper_edge_index_1,
                            hyper_micro_edge_index_0, hyper_micro_edge_index_1]):
        node, he = _chunked(ei[0]), _chunked(ei[1])
        pm = _spmm(xt, he, node, zz)
        m = _scale2(pm, binv[i])
        outs.append(_spmm(m, node, he, zz))

    h_emb = _fusion(outs[0], dinv[0], outs[1], dinv[1],
                    hg_bias, fus_W1, fus_b1, fus_W2, fus_b2)
    h_emb_micro = _fusion(outs[2], dinv[2], outs[3], dinv[3],
                          hg_bias, fus_W1, fus_b1, fus_W2, fus_b2)

    return _attention(u0[:N], S[:N], h_emb, h_emb_micro, att, att_m)


# speedup vs baseline: 1.1022x; 1.0137x over previous
import functools

import jax
import jax.numpy as jnp
from jax import lax
from jax.experimental import pallas as pl
from jax.experimental.pallas import tpu as pltpu
from jax.experimental.pallas import tpu_sc as plsc

N = 10000
D = 64
E = 160000

NC = 2
NS = 16
NW = NC * NS
EPT = E // NW
CH = 128
NCHT = E // CH
CPT = 40
NCHP = NW * CPT
NP = 10240
RPT = NP // NS
HI = jax.lax.Precision.HIGHEST

_mesh = lambda: plsc.VectorSubcoreMesh(core_axis_name="c", subcore_axis_name="s")


def _degrees(f_heter, f_h0, f_h1, f_hm0, f_hm1):
    @functools.partial(
        pl.kernel,
        out_type=jax.ShapeDtypeStruct((NW * 9 * N,), jnp.float32),
        mesh=_mesh(),
        compiler_params=pltpu.CompilerParams(needs_layout_passes=False),
        scratch_types=[
            pltpu.VMEM((9 * N,), jnp.float32),
            pltpu.VMEM((EPT + 16,), jnp.int32),
        ],
    )
    def k(heter_h, h0_h, h1_h, hm0_h, hm1_h, out_h, hist_v, idx_v):
        cid = lax.axis_index("c")
        sid = lax.axis_index("s")
        wid = cid * NS + sid
        zeros16 = jnp.zeros((16,), jnp.float32)

        def zb(i, _):
            hist_v[pl.ds(i * 16, 16)] = zeros16
            return 0

        lax.fori_loop(0, (9 * N) // 16, zb, 0)

        base_e = wid * EPT
        ones = jnp.ones((16,), jnp.float32)
        tail_mask = lax.iota(jnp.int32, 16) < (EPT - (EPT // 16) * 16)
        jobs = [
            (heter_h, 0, 0),
            (h0_h, 0, 1), (h0_h, 1, 2),
            (h1_h, 0, 3), (h1_h, 1, 4),
            (hm0_h, 0, 5), (hm0_h, 1, 6),
            (hm1_h, 0, 7), (hm1_h, 1, 8),
        ]
        for src, row, j in jobs:
            pltpu.sync_copy(src.at[pl.ds(row * E + base_e, EPT)],
                            idx_v.at[pl.ds(0, EPT)])
            off = jnp.int32(j * N)

            def body(i, _):
                ix = idx_v[pl.ds(i * 16, 16)] + off
                plsc.addupdate_scatter(hist_v, [ix], ones)
                return 0

            lax.fori_loop(0, EPT // 16, body, 0)
            ixt = idx_v[pl.ds((EPT // 16) * 16, 16)] + off
            plsc.addupdate_scatter(hist_v, [ixt], ones, mask=tail_mask)

        pltpu.sync_copy(hist_v, out_h.at[pl.ds(wid * 9 * N, 9 * N)])

    return k(f_heter, f_h0, f_h1, f_hm0, f_hm1)


def _spmm(z, src2, dst2, zeros_stripe, ap=NP, cpt=CPT, ncht=NCHT, dep=4):
    @functools.partial(
        pl.kernel,
        out_type=jax.ShapeDtypeStruct((2 * ap, D), jnp.float32),
        mesh=_mesh(),
        compiler_params=pltpu.CompilerParams(needs_layout_passes=False,
                                             use_tc_tiling_on_sc=False),
        scratch_types=[
            pltpu.VMEM_SHARED((ap, D), jnp.float32),
            [pltpu.VMEM((CH, D), jnp.float32)] * dep,
            pltpu.VMEM((cpt, CH), jnp.int32),
            pltpu.VMEM((cpt, CH), jnp.int32),
            [pltpu.SemaphoreType.DMA] * dep,
            [pltpu.SemaphoreType.DMA] * dep,
        ],
    )
    def k(z_h, src_h, dst_h, zz_h, out_h,
          acc_s, rows, si_big, gi_big, sg, ss):
        rpt = ap // NS
        cid = lax.axis_index("c")
        sid = lax.axis_index("s")
        wid = cid * NS + sid
        row0 = sid * rpt

        pltpu.sync_copy(zz_h, acc_s.at[pl.ds(row0, rpt)])

        r0 = wid * cpt
        pltpu.sync_copy(dst_h.at[pl.ds(r0, cpt)], gi_big)
        pltpu.sync_copy(src_h.at[pl.ds(r0, cpt)], si_big)
        nc = jnp.where(wid == NW - 1, ncht - (NW - 1) * cpt, cpt)

        plsc.subcore_barrier()

        def start_gather(j, b):
            pltpu.async_copy(z_h.at[gi_big.at[j]], rows[b], sg[b])

        def wait_gather(b):
            pltpu.make_async_copy(z_h.at[gi_big.at[0]], rows[b], sg[b]).wait()

        def start_scatter(j, b):
            pltpu.async_copy(rows[b], acc_s.at[si_big.at[j]], ss[b], add=True)

        def wait_scatter(b):
            pltpu.make_async_copy(rows[b], acc_s.at[si_big.at[0]], ss[b]).wait()

        for b in range(dep):
            start_gather(b, b)

        def quad(q, _):
            j0 = dep * q
            for b in range(dep):
                wait_gather(b)
                start_scatter(j0 + b, b)
            for b in range(dep):
                wait_scatter(b)
                start_gather(j0 + dep + b, b)
            return 0

        lax.fori_loop(0, (nc - dep) // dep, quad, 0)

        jl = dep * ((nc - dep) // dep)
        for b in range(dep):
            wait_gather(b)
            start_scatter(jl + b, b)
        for b in range(dep):
            wait_scatter(b)

        def seq(r, _):
            j = jl + dep + r
            pltpu.async_copy(z_h.at[gi_big.at[j]], rows[0], sg[0])
            wait_gather(0)
            pltpu.sync_copy(rows[0], acc_s.at[si_big.at[j]], add=True)
            return 0

        lax.fori_loop(0, nc - jl - dep, seq, 0)

        plsc.subcore_barrier()
        pltpu.sync_copy(acc_s.at[pl.ds(row0, rpt)],
                        out_h.at[pl.ds(cid * ap + row0, rpt)])

    return k(z, src2, dst2, zeros_stripe)


_R = 1000
_G = N // _R
_RP = 1280
_GP = NP // _RP


def _row_spec(shape_tail):
    return pl.BlockSpec((_R,) + shape_tail, lambda i: (i,) + (0,) * len(shape_tail))


def _rowp_spec(shape_tail):
    return pl.BlockSpec((_RP,) + shape_tail, lambda i: (i,) + (0,) * len(shape_tail))


def _full_spec(shape):
    return pl.BlockSpec(shape, lambda i: (0,) * len(shape))


def _prep(emb, W0, b0, W1, b1, theta):
    def body(e_r, w0_r, b0_r, w1_r, b1_r, th_r, u0_r, c_r, xt_r):
        e = e_r[...]
        u0 = e * jax.nn.sigmoid(jnp.dot(e, w0_r[...], precision=HI) + b0_r[...])
        c = e * jax.nn.sigmoid(jnp.dot(e, w1_r[...], precision=HI) + b1_r[...])
        u0_r[...] = u0
        c_r[...] = c
        xt_r[...] = jnp.dot(c, th_r[...], precision=HI)

    o = jax.ShapeDtypeStruct((NP, D), jnp.float32)
    return pl.pallas_call(
        body,
        grid=(_GP,),
        in_specs=[_rowp_spec((D,)), _full_spec((D, D)), _full_spec((1, D)),
                  _full_spec((D, D)), _full_spec((1, D)), _full_spec((D, D))],
        out_specs=[_rowp_spec((D,))] * 3,
        out_shape=[o, o, o],
    )(emb, W0, b0, W1, b1, theta)


def _factors(per_tile):
    def body(h_r, out_r):
        c = jnp.sum(h_r[...], axis=0)
        deg = c[0:1]
        dsi = lax.rsqrt(jnp.where(deg == 0.0, 1.0, deg))
        inv = jnp.where(c[1:9] > 0.0, 1.0 / jnp.where(c[1:9] > 0.0, c[1:9], 1.0), 0.0)
        out_r[...] = jnp.concatenate([dsi, inv], axis=0)

    return pl.pallas_call(
        body,
        grid=(1,),
        in_specs=[_full_spec((NW, 9, N))],
        out_specs=_full_spec((9, N)),
        out_shape=jax.ShapeDtypeStruct((9, N), jnp.float32),
    )(per_tile)


def _norm_scale(x, dsi):
    def body(x_r, d_r, z_r):
        x = x_r[...]
        nrm = jnp.sqrt(jnp.sum(x * x, axis=1, keepdims=True)) + 1e-12
        z_r[...] = x * d_r[...] / nrm

    return pl.pallas_call(
        body,
        grid=(_GP,),
        in_specs=[_rowp_spec((D,)), _rowp_spec((1,))],
        out_specs=_rowp_spec((D,)),
        out_shape=jax.ShapeDtypeStruct((NP, D), jnp.float32),
    )(x, dsi)


def _combine(p, dsi, S):
    def body(p0_r, p1_r, d_r, s_r, z_r, so_r):
        d = d_r[...]
        x = (p0_r[...] + p1_r[...]) * d
        so_r[...] = s_r[...] + x
        nrm = jnp.sqrt(jnp.sum(x * x, axis=1, keepdims=True)) + 1e-12
        z_r[...] = x * d / nrm

    o = jax.ShapeDtypeStruct((NP, D), jnp.float32)
    return pl.pallas_call(
        body,
        grid=(_GP,),
        in_specs=[_rowp_spec((D,))] * 2 + [_rowp_spec((1,)), _rowp_spec((D,))],
        out_specs=[_rowp_spec((D,))] * 2,
        out_shape=[o, o],
    )(p[:NP], p[NP:], dsi, S)


def _scale2(p0, p1, binv):
    def body(p0_r, p1_r, b_r, m_r):
        m_r[...] = (p0_r[...] + p1_r[...]) * b_r[...]

    return pl.pallas_call(
        body,
        grid=(_GP,),
        in_specs=[_rowp_spec((D,))] * 2 + [_rowp_spec((1,))],
        out_specs=_rowp_spec((D,)),
        out_shape=jax.ShapeDtypeStruct((NP, D), jnp.float32),
    )(p0, p1, binv)


def _gelu(x):
    return 0.5 * x * (1.0 + lax.erf(x * (2.0 ** -0.5)))


def _fusion(q00, q01, dinv0, q10, q11, dinv1, hg_bias, fW1, fb1, fW2, fb2):
    def body(q00_r, q01_r, d0_r, q10_r, q11_r, d1_r, bias_r,
             w1_r, b1_r, w2_r, out_r):
        bias = bias_r[...]
        h0 = (q00_r[...] + q01_r[...]) * d0_r[...] + bias
        h1 = (q10_r[...] + q11_r[...]) * d1_r[...] + bias
        w1t = w1_r[...]
        dn = (((1,), (1,)), ((), ()))
        g0 = _gelu(lax.dot_general(h0, w1t, dn, precision=HI) + b1_r[...])
        g1 = _gelu(lax.dot_general(h1, w1t, dn, precision=HI) + b1_r[...])
        w2 = w2_r[...]
        s0 = lax.dot_general(g0, w2, dn, precision=HI)
        s1 = lax.dot_general(g1, w2, dn, precision=HI)
        mx = jnp.maximum(s0, s1)
        e0 = jnp.exp(s0 - mx)
        e1 = jnp.exp(s1 - mx)
        out_r[...] = (e0 * h0 + e1 * h1) / (e0 + e1)

    return pl.pallas_call(
        body,
        grid=(_G,),
        in_specs=[_row_spec((D,)), _row_spec((D,)), _row_spec((1,)),
                  _row_spec((D,)), _row_spec((D,)), _row_spec((1,)),
                  _full_spec((1, D)),
                  _full_spec((D, D)), _full_spec((1, D)),
                  _full_spec((1, D))],
        out_specs=_row_spec((D,)),
        out_shape=jax.ShapeDtypeStruct((N, D), jnp.float32),
    )(q00, q01, dinv0, q10, q11, dinv1,
      hg_bias.reshape(1, D), fW1, fb1.reshape(1, D), fW2)


def _attention(u0, S, h_emb, hm_emb, att, att_m):
    def body(u0_r, s_r, h_r, hm_r, att_r, am_r, out_r):
        u = 0.1 * u0_r[...] + 0.15 * s_r[...]
        h = h_r[...]
        hm = hm_r[...]
        dn = (((1,), (1,)), ((), ()))
        v = lax.dot_general(am_r[...], att_r[...], dn, precision=HI)
        dn2 = (((1,), (0,)), ((), ()))
        wu = lax.dot_general(u, v, dn2, precision=HI)
        wh = lax.dot_general(h, v, dn2, precision=HI)
        wm = lax.dot_general(hm, v, dn2, precision=HI)
        mx = jnp.maximum(jnp.maximum(wu, wh), wm)
        eu = jnp.exp(wu - mx)
        eh = jnp.exp(wh - mx)
        em = jnp.exp(wm - mx)
        out_r[...] = (eu * u + eh * h + em * hm) / (eu + eh + em)

    return pl.pallas_call(
        body,
        grid=(_G,),
        in_specs=[_row_spec((D,))] * 4 + [_full_spec((1, D)), _full_spec((D, D))],
        out_specs=_row_spec((D,)),
        out_shape=jax.ShapeDtypeStruct((N, D), jnp.float32),
    )(u0, S, h_emb, hm_emb, att, att_m)


def kernel(heter_edge_index, hyper_edge_index_0, hyper_edge_index_1,
           hyper_micro_edge_index_0, hyper_micro_edge_index_1, emb_table,
           W0, b0, W1, b1, theta, hg_bias, att, att_m,
           fus_W1, fus_b1, fus_W2, fus_b2):
    f_heter = heter_edge_index.reshape(-1)
    f_h0 = hyper_edge_index_0.reshape(-1)
    f_h1 = hyper_edge_index_1.reshape(-1)
    f_hm0 = hyper_micro_edge_index_0.reshape(-1)
    f_hm1 = hyper_micro_edge_index_1.reshape(-1)

    hist = _degrees(f_heter, f_h0, f_h1, f_hm0, f_hm1)
    fac = _factors(hist.reshape(NW, 9, N))
    padc = jnp.zeros((NP - N, 1), jnp.float32)

    def _padded(row):
        return jnp.concatenate([row.reshape(N, 1), padc], axis=0)

    dsi = _padded(fac[0])
    dinv = [fac[i].reshape(N, 1) for i in (1, 3, 5, 7)]
    binv = [_padded(fac[i]) for i in (2, 4, 6, 8)]

    emb_p = jnp.concatenate([emb_table, jnp.zeros((NP - N, D), jnp.float32)], axis=0)
    u0, c_emb, xt = _prep(emb_p, W0, b0, W1, b1, theta)
    del c_emb

    zz = jnp.zeros((RPT, D), jnp.float32)

    def _chunked(ix):
        pad = jnp.zeros((NCHP - NCHT, CH), jnp.int32)
        return jnp.concatenate([ix.reshape(NCHT, CH), pad], axis=0)

    z = _norm_scale(u0, dsi)
    S = jnp.zeros((NP, D), jnp.float32)
    src = _chunked(heter_edge_index[0])
    dst = _chunked(heter_edge_index[1])
    for _ in range(6):
        p = _spmm(z, src, dst, zz)
        z, S = _combine(p, dsi, S)

    NCHT2 = 2 * E // CH
    CPT2 = 80
    zz2 = jnp.zeros((2 * NP // NS, D), jnp.float32)

    def _chunked2(ix):
        pad2 = jnp.zeros((NW * CPT2 - NCHT2, CH), jnp.int32)
        return jnp.concatenate([ix.reshape(NCHT2, CH), pad2], axis=0)

    def _conv_pair(ei_a, ei_b):
        node = jnp.concatenate([ei_a[0], ei_b[0]])
        he_ofs = jnp.concatenate([ei_a[1], ei_b[1] + NP])
        return _chunked2(he_ofs), _chunked2(node)

    h_out = []
    for ei_a, ei_b, bi_a, bi_b in (
            (hyper_edge_index_0, hyper_edge_index_1, binv[0], binv[1]),
            (hyper_micro_edge_index_0, hyper_micro_edge_index_1, binv[2], binv[3])):
        he2, node2 = _conv_pair(ei_a, ei_b)
        pm = _spmm(xt, he2, node2, zz2, ap=2 * NP, cpt=CPT2, ncht=NCHT2, dep=3)
        m_a = _scale2(pm[0:NP], pm[2 * NP:3 * NP], bi_a)
        m_b = _scale2(pm[NP:2 * NP], pm[3 * NP:4 * NP], bi_b)
        z2 = jnp.concatenate([m_a, m_b], axis=0)
        po = _spmm(z2, node2, he2, zz2, ap=2 * NP, cpt=CPT2, ncht=NCHT2, dep=3)
        h_out.append(po)

    h_emb = _fusion(h_out[0][0:N], h_out[0][2 * NP:2 * NP + N], dinv[0],
                    h_out[0][NP:NP + N], h_out[0][3 * NP:3 * NP + N], dinv[1],
                    hg_bias, fus_W1, fus_b1, fus_W2, fus_b2)
    h_emb_micro = _fusion(h_out[1][0:N], h_out[1][2 * NP:2 * NP + N], dinv[2],
                          h_out[1][NP:NP + N], h_out[1][3 * NP:3 * NP + N], dinv[3],
                          hg_bias, fus_W1, fus_b1, fus_W2, fus_b2)

    return _attention(u0[:N], S[:N], h_emb, h_emb_micro, att, att_m)


# speedup vs baseline: 1.1225x; 1.0184x over previous
import functools

import jax
import jax.numpy as jnp
from jax import lax
from jax.experimental import pallas as pl
from jax.experimental.pallas import tpu as pltpu
from jax.experimental.pallas import tpu_sc as plsc

N = 10000
D = 64
E = 160000

NC = 2
NS = 16
NW = NC * NS
EPT = E // NW
CH = 128
NCHT = E // CH
CPT = 40
NCHP = NW * CPT
NP = 10240
RPT = NP // NS
HI = jax.lax.Precision.HIGHEST

_mesh = lambda: plsc.VectorSubcoreMesh(core_axis_name="c", subcore_axis_name="s")


def _degrees(f_heter, f_h0, f_h1, f_hm0, f_hm1):
    @functools.partial(
        pl.kernel,
        out_type=jax.ShapeDtypeStruct((NW * 9 * N,), jnp.float32),
        mesh=_mesh(),
        compiler_params=pltpu.CompilerParams(needs_layout_passes=False),
        scratch_types=[
            pltpu.VMEM((9 * N,), jnp.float32),
            pltpu.VMEM((EPT + 16,), jnp.int32),
            pltpu.VMEM((EPT + 16,), jnp.int32),
            pltpu.SemaphoreType.DMA,
        ],
    )
    def k(heter_h, h0_h, h1_h, hm0_h, hm1_h, out_h, hist_v, idx_a, idx_b, sem):
        cid = lax.axis_index("c")
        sid = lax.axis_index("s")
        wid = cid * NS + sid
        zeros16 = jnp.zeros((16,), jnp.float32)

        def zb(i, _):
            for kq in range(9):
                hist_v[pl.ds(i * 144 + kq * 16, 16)] = zeros16
            return 0

        lax.fori_loop(0, (9 * N) // 144, zb, 0)

        base_e = wid * EPT
        ones = jnp.ones((16,), jnp.float32)
        tail_mask = lax.iota(jnp.int32, 16) < (EPT - (EPT // 16) * 16)
        jobs = [
            (heter_h, 0, 0),
            (h0_h, 0, 1), (h0_h, 1, 2),
            (h1_h, 0, 3), (h1_h, 1, 4),
            (hm0_h, 0, 5), (hm0_h, 1, 6),
            (hm1_h, 0, 7), (hm1_h, 1, 8),
        ]
        bufs = [idx_a, idx_b]
        pltpu.async_copy(jobs[0][0].at[pl.ds(jobs[0][1] * E + base_e, EPT)],
                         bufs[0].at[pl.ds(0, EPT)], sem)
        for jn, (src, row, j) in enumerate(jobs):
            buf = bufs[jn % 2]
            pltpu.make_async_copy(src.at[pl.ds(row * E + base_e, EPT)],
                                  buf.at[pl.ds(0, EPT)], sem).wait()
            if jn + 1 < len(jobs):
                nsrc, nrow, _ = jobs[jn + 1]
                pltpu.async_copy(nsrc.at[pl.ds(nrow * E + base_e, EPT)],
                                 bufs[(jn + 1) % 2].at[pl.ds(0, EPT)], sem)
            off = jnp.int32(j * N)

            def body(i, _):
                for kq in range(4):
                    ix = buf[pl.ds(i * 64 + kq * 16, 16)] + off
                    plsc.addupdate_scatter(hist_v, [ix], ones)
                return 0

            lax.fori_loop(0, EPT // 64, body, 0)
            ixt = buf[pl.ds((EPT // 16) * 16, 16)] + off
            plsc.addupdate_scatter(hist_v, [ixt], ones, mask=tail_mask)

        pltpu.sync_copy(hist_v, out_h.at[pl.ds(wid * 9 * N, 9 * N)])

    return k(f_heter, f_h0, f_h1, f_hm0, f_hm1)


def _spmm(z, src2, dst2, zeros_stripe, ap=NP, cpt=CPT, ncht=NCHT, dep=4):
    @functools.partial(
        pl.kernel,
        out_type=jax.ShapeDtypeStruct((2 * ap, D), jnp.float32),
        mesh=_mesh(),
        compiler_params=pltpu.CompilerParams(needs_layout_passes=False,
                                             use_tc_tiling_on_sc=False),
        scratch_types=[
            pltpu.VMEM_SHARED((ap, D), jnp.float32),
            [pltpu.VMEM((CH, D), jnp.float32)] * dep,
            pltpu.VMEM((cpt, CH), jnp.int32),
            pltpu.VMEM((cpt, CH), jnp.int32),
            [pltpu.SemaphoreType.DMA] * dep,
            [pltpu.SemaphoreType.DMA] * dep,
        ],
    )
    def k(z_h, src_h, dst_h, zz_h, out_h,
          acc_s, rows, si_big, gi_big, sg, ss):
        rpt = ap // NS
        cid = lax.axis_index("c")
        sid = lax.axis_index("s")
        wid = cid * NS + sid
        row0 = sid * rpt

        pltpu.sync_copy(zz_h, acc_s.at[pl.ds(row0, rpt)])

        r0 = wid * cpt
        pltpu.sync_copy(dst_h.at[pl.ds(r0, cpt)], gi_big)
        pltpu.sync_copy(src_h.at[pl.ds(r0, cpt)], si_big)
        nc = jnp.where(wid == NW - 1, ncht - (NW - 1) * cpt, cpt)

        plsc.subcore_barrier()

        def start_gather(j, b):
            pltpu.async_copy(z_h.at[gi_big.at[j]], rows[b], sg[b])

        def wait_gather(b):
            pltpu.make_async_copy(z_h.at[gi_big.at[0]], rows[b], sg[b]).wait()

        def start_scatter(j, b):
            pltpu.async_copy(rows[b], acc_s.at[si_big.at[j]], ss[b], add=True)

        def wait_scatter(b):
            pltpu.make_async_copy(rows[b], acc_s.at[si_big.at[0]], ss[b]).wait()

        for b in range(dep):
            start_gather(b, b)

        def quad(q, _):
            j0 = dep * q
            for b in range(dep):
                wait_gather(b)
                start_scatter(j0 + b, b)
            for b in range(dep):
                wait_scatter(b)
                start_gather(j0 + dep + b, b)
            return 0

        lax.fori_loop(0, (nc - dep) // dep, quad, 0)

        jl = dep * ((nc - dep) // dep)
        for b in range(dep):
            wait_gather(b)
            start_scatter(jl + b, b)
        for b in range(dep):
            wait_scatter(b)

        def seq(r, _):
            j = jl + dep + r
            pltpu.async_copy(z_h.at[gi_big.at[j]], rows[0], sg[0])
            wait_gather(0)
            pltpu.sync_copy(rows[0], acc_s.at[si_big.at[j]], add=True)
            return 0

        lax.fori_loop(0, nc - jl - dep, seq, 0)

        plsc.subcore_barrier()
        pltpu.sync_copy(acc_s.at[pl.ds(row0, rpt)],
                        out_h.at[pl.ds(cid * ap + row0, rpt)])

    return k(z, src2, dst2, zeros_stripe)


_R = 1000
_G = N // _R
_RP = 1280
_GP = NP // _RP


def _row_spec(shape_tail):
    return pl.BlockSpec((_R,) + shape_tail, lambda i: (i,) + (0,) * len(shape_tail))


def _rowp_spec(shape_tail):
    return pl.BlockSpec((_RP,) + shape_tail, lambda i: (i,) + (0,) * len(shape_tail))


def _full_spec(shape):
    return pl.BlockSpec(shape, lambda i: (0,) * len(shape))


def _prep(emb, W0, b0, W1, b1, theta):
    def body(e_r, w0_r, b0_r, w1_r, b1_r, th_r, u0_r, c_r, xt_r):
        e = e_r[...]
        u0 = e * jax.nn.sigmoid(jnp.dot(e, w0_r[...], precision=HI) + b0_r[...])
        c = e * jax.nn.sigmoid(jnp.dot(e, w1_r[...], precision=HI) + b1_r[...])
        u0_r[...] = u0
        c_r[...] = c
        xt_r[...] = jnp.dot(c, th_r[...], precision=HI)

    o = jax.ShapeDtypeStruct((NP, D), jnp.float32)
    return pl.pallas_call(
        body,
        grid=(_GP,),
        in_specs=[_rowp_spec((D,)), _full_spec((D, D)), _full_spec((1, D)),
                  _full_spec((D, D)), _full_spec((1, D)), _full_spec((D, D))],
        out_specs=[_rowp_spec((D,))] * 3,
        out_shape=[o, o, o],
    )(emb, W0, b0, W1, b1, theta)


def _factors(per_tile):
    def body(h_r, out_r):
        c = jnp.sum(h_r[...], axis=0)
        deg = c[0:1]
        dsi = lax.rsqrt(jnp.where(deg == 0.0, 1.0, deg))
        inv = jnp.where(c[1:9] > 0.0, 1.0 / jnp.where(c[1:9] > 0.0, c[1:9], 1.0), 0.0)
        out_r[...] = jnp.concatenate([dsi, inv], axis=0)

    return pl.pallas_call(
        body,
        grid=(1,),
        in_specs=[_full_spec((NW, 9, N))],
        out_specs=_full_spec((9, N)),
        out_shape=jax.ShapeDtypeStruct((9, N), jnp.float32),
    )(per_tile)


def _norm_scale(x, dsi):
    def body(x_r, d_r, z_r):
        x = x_r[...]
        nrm = jnp.sqrt(jnp.sum(x * x, axis=1, keepdims=True)) + 1e-12
        z_r[...] = x * d_r[...] / nrm

    return pl.pallas_call(
        body,
        grid=(_GP,),
        in_specs=[_rowp_spec((D,)), _rowp_spec((1,))],
        out_specs=_rowp_spec((D,)),
        out_shape=jax.ShapeDtypeStruct((NP, D), jnp.float32),
    )(x, dsi)


def _combine(p, dsi, S):
    def body(p0_r, p1_r, d_r, s_r, z_r, so_r):
        d = d_r[...]
        x = (p0_r[...] + p1_r[...]) * d
        so_r[...] = s_r[...] + x
        nrm = jnp.sqrt(jnp.sum(x * x, axis=1, keepdims=True)) + 1e-12
        z_r[...] = x * d / nrm

    o = jax.ShapeDtypeStruct((NP, D), jnp.float32)
    return pl.pallas_call(
        body,
        grid=(_GP,),
        in_specs=[_rowp_spec((D,))] * 2 + [_rowp_spec((1,)), _rowp_spec((D,))],
        out_specs=[_rowp_spec((D,))] * 2,
        out_shape=[o, o],
    )(p[:NP], p[NP:], dsi, S)


def _scale2(p0, p1, binv):
    def body(p0_r, p1_r, b_r, m_r):
        m_r[...] = (p0_r[...] + p1_r[...]) * b_r[...]

    return pl.pallas_call(
        body,
        grid=(_GP,),
        in_specs=[_rowp_spec((D,))] * 2 + [_rowp_spec((1,))],
        out_specs=_rowp_spec((D,)),
        out_shape=jax.ShapeDtypeStruct((NP, D), jnp.float32),
    )(p0, p1, binv)


def _gelu(x):
    return 0.5 * x * (1.0 + lax.erf(x * (2.0 ** -0.5)))


def _fusion(q00, q01, dinv0, q10, q11, dinv1, hg_bias, fW1, fb1, fW2, fb2):
    def body(q00_r, q01_r, d0_r, q10_r, q11_r, d1_r, bias_r,
             w1_r, b1_r, w2_r, out_r):
        bias = bias_r[...]
        h0 = (q00_r[...] + q01_r[...]) * d0_r[...] + bias
        h1 = (q10_r[...] + q11_r[...]) * d1_r[...] + bias
        w1t = w1_r[...]
        dn = (((1,), (1,)), ((), ()))
        g0 = _gelu(lax.dot_general(h0, w1t, dn, precision=HI) + b1_r[...])
        g1 = _gelu(lax.dot_general(h1, w1t, dn, precision=HI) + b1_r[...])
        w2 = w2_r[...]
        s0 = lax.dot_general(g0, w2, dn, precision=HI)
        s1 = lax.dot_general(g1, w2, dn, precision=HI)
        mx = jnp.maximum(s0, s1)
        e0 = jnp.exp(s0 - mx)
        e1 = jnp.exp(s1 - mx)
        out_r[...] = (e0 * h0 + e1 * h1) / (e0 + e1)

    return pl.pallas_call(
        body,
        grid=(_G,),
        in_specs=[_row_spec((D,)), _row_spec((D,)), _row_spec((1,)),
                  _row_spec((D,)), _row_spec((D,)), _row_spec((1,)),
                  _full_spec((1, D)),
                  _full_spec((D, D)), _full_spec((1, D)),
                  _full_spec((1, D))],
        out_specs=_row_spec((D,)),
        out_shape=jax.ShapeDtypeStruct((N, D), jnp.float32),
    )(q00, q01, dinv0, q10, q11, dinv1,
      hg_bias.reshape(1, D), fW1, fb1.reshape(1, D), fW2)


def _attention(u0, S, h_emb, hm_emb, att, att_m):
    def body(u0_r, s_r, h_r, hm_r, att_r, am_r, out_r):
        u = 0.1 * u0_r[...] + 0.15 * s_r[...]
        h = h_r[...]
        hm = hm_r[...]
        dn = (((1,), (1,)), ((), ()))
        v = lax.dot_general(am_r[...], att_r[...], dn, precision=HI)
        dn2 = (((1,), (0,)), ((), ()))
        wu = lax.dot_general(u, v, dn2, precision=HI)
        wh = lax.dot_general(h, v, dn2, precision=HI)
        wm = lax.dot_general(hm, v, dn2, precision=HI)
        mx = jnp.maximum(jnp.maximum(wu, wh), wm)
        eu = jnp.exp(wu - mx)
        eh = jnp.exp(wh - mx)
        em = jnp.exp(wm - mx)
        out_r[...] = (eu * u + eh * h + em * hm) / (eu + eh + em)

    return pl.pallas_call(
        body,
        grid=(_G,),
        in_specs=[_row_spec((D,))] * 4 + [_full_spec((1, D)), _full_spec((D, D))],
        out_specs=_row_spec((D,)),
        out_shape=jax.ShapeDtypeStruct((N, D), jnp.float32),
    )(u0, S, h_emb, hm_emb, att, att_m)


def kernel(heter_edge_index, hyper_edge_index_0, hyper_edge_index_1,
           hyper_micro_edge_index_0, hyper_micro_edge_index_1, emb_table,
           W0, b0, W1, b1, theta, hg_bias, att, att_m,
           fus_W1, fus_b1, fus_W2, fus_b2):
    f_heter = heter_edge_index.reshape(-1)
    f_h0 = hyper_edge_index_0.reshape(-1)
    f_h1 = hyper_edge_index_1.reshape(-1)
    f_hm0 = hyper_micro_edge_index_0.reshape(-1)
    f_hm1 = hyper_micro_edge_index_1.reshape(-1)

    hist = _degrees(f_heter, f_h0, f_h1, f_hm0, f_hm1)
    fac = _factors(hist.reshape(NW, 9, N))
    padc = jnp.zeros((NP - N, 1), jnp.float32)

    def _padded(row):
        return jnp.concatenate([row.reshape(N, 1), padc], axis=0)

    dsi = _padded(fac[0])
    dinv = [fac[i].reshape(N, 1) for i in (1, 3, 5, 7)]
    binv = [_padded(fac[i]) for i in (2, 4, 6, 8)]

    emb_p = jnp.concatenate([emb_table, jnp.zeros((NP - N, D), jnp.float32)], axis=0)
    u0, c_emb, xt = _prep(emb_p, W0, b0, W1, b1, theta)
    del c_emb

    zz = jnp.zeros((RPT, D), jnp.float32)

    def _chunked(ix):
        pad = jnp.zeros((NCHP - NCHT, CH), jnp.int32)
        return jnp.concatenate([ix.reshape(NCHT, CH), pad], axis=0)

    z = _norm_scale(u0, dsi)
    S = jnp.zeros((NP, D), jnp.float32)
    src = _chunked(heter_edge_index[0])
    dst = _chunked(heter_edge_index[1])
    for _ in range(6):
        p = _spmm(z, src, dst, zz)
        z, S = _combine(p, dsi, S)

    NCHT2 = 2 * E // CH
    CPT2 = 80
    zz2 = jnp.zeros((2 * NP // NS, D), jnp.float32)

    def _chunked2(ix):
        pad2 = jnp.zeros((NW * CPT2 - NCHT2, CH), jnp.int32)
        return jnp.concatenate([ix.reshape(NCHT2, CH), pad2], axis=0)

    def _conv_pair(ei_a, ei_b):
        node = jnp.concatenate([ei_a[0], ei_b[0]])
        he_ofs = jnp.concatenate([ei_a[1], ei_b[1] + NP])
        return _chunked2(he_ofs), _chunked2(node)

    h_out = []
    for ei_a, ei_b, bi_a, bi_b in (
            (hyper_edge_index_0, hyper_edge_index_1, binv[0], binv[1]),
            (hyper_micro_edge_index_0, hyper_micro_edge_index_1, binv[2], binv[3])):
        he2, node2 = _conv_pair(ei_a, ei_b)
        pm = _spmm(xt, he2, node2, zz2, ap=2 * NP, cpt=CPT2, ncht=NCHT2, dep=3)
        m_a = _scale2(pm[0:NP], pm[2 * NP:3 * NP], bi_a)
        m_b = _scale2(pm[NP:2 * NP], pm[3 * NP:4 * NP], bi_b)
        z2 = jnp.concatenate([m_a, m_b], axis=0)
        po = _spmm(z2, node2, he2, zz2, ap=2 * NP, cpt=CPT2, ncht=NCHT2, dep=3)
        h_out.append(po)

    h_emb = _fusion(h_out[0][0:N], h_out[0][2 * NP:2 * NP + N], dinv[0],
                    h_out[0][NP:NP + N], h_out[0][3 * NP:3 * NP + N], dinv[1],
                    hg_bias, fus_W1, fus_b1, fus_W2, fus_b2)
    h_emb_micro = _fusion(h_out[1][0:N], h_out[1][2 * NP:2 * NP + N], dinv[2],
                          h_out[1][NP:NP + N], h_out[1][3 * NP:3 * NP + N], dinv[3],
                          hg_bias, fus_W1, fus_b1, fus_W2, fus_b2)

    return _attention(u0[:N], S[:N], h_emb, h_emb_micro, att, att_m)


# speedup vs baseline: 1.1800x; 1.0512x over previous
import functools

import jax
import jax.numpy as jnp
from jax import lax
from jax.experimental import pallas as pl
from jax.experimental.pallas import tpu as pltpu
from jax.experimental.pallas import tpu_sc as plsc

N = 10000
D = 64
E = 160000

NC = 2
NS = 16
NW = NC * NS
EPT = E // NW
CH = 128
NCHT = E // CH
CPT = 40
NCHP = NW * CPT
NP = 10240
RPT = NP // NS
HI = jax.lax.Precision.HIGHEST

_mesh = lambda: plsc.VectorSubcoreMesh(core_axis_name="c", subcore_axis_name="s")


def _degrees(f_heter, f_h0, f_h1, f_hm0, f_hm1):
    @functools.partial(
        pl.kernel,
        out_type=jax.ShapeDtypeStruct((NW * 9 * N,), jnp.float32),
        mesh=_mesh(),
        compiler_params=pltpu.CompilerParams(needs_layout_passes=False),
        scratch_types=[
            pltpu.VMEM((9 * N,), jnp.float32),
            pltpu.VMEM((EPT + 16,), jnp.int32),
            pltpu.VMEM((EPT + 16,), jnp.int32),
            pltpu.SemaphoreType.DMA,
        ],
    )
    def k(heter_h, h0_h, h1_h, hm0_h, hm1_h, out_h, hist_v, idx_a, idx_b, sem):
        cid = lax.axis_index("c")
        sid = lax.axis_index("s")
        wid = cid * NS + sid
        zeros16 = jnp.zeros((16,), jnp.float32)

        def zb(i, _):
            for kq in range(9):
                hist_v[pl.ds(i * 144 + kq * 16, 16)] = zeros16
            return 0

        lax.fori_loop(0, (9 * N) // 144, zb, 0)

        base_e = wid * EPT
        ones = jnp.ones((16,), jnp.float32)
        tail_mask = lax.iota(jnp.int32, 16) < (EPT - (EPT // 16) * 16)
        jobs = [
            (heter_h, 0, 0),
            (h0_h, 0, 1), (h0_h, 1, 2),
            (h1_h, 0, 3), (h1_h, 1, 4),
            (hm0_h, 0, 5), (hm0_h, 1, 6),
            (hm1_h, 0, 7), (hm1_h, 1, 8),
        ]
        bufs = [idx_a, idx_b]
        pltpu.async_copy(jobs[0][0].at[pl.ds(jobs[0][1] * E + base_e, EPT)],
                         bufs[0].at[pl.ds(0, EPT)], sem)
        for jn, (src, row, j) in enumerate(jobs):
            buf = bufs[jn % 2]
            pltpu.make_async_copy(src.at[pl.ds(row * E + base_e, EPT)],
                                  buf.at[pl.ds(0, EPT)], sem).wait()
            if jn + 1 < len(jobs):
                nsrc, nrow, _ = jobs[jn + 1]
                pltpu.async_copy(nsrc.at[pl.ds(nrow * E + base_e, EPT)],
                                 bufs[(jn + 1) % 2].at[pl.ds(0, EPT)], sem)
            off = jnp.int32(j * N)

            def body(i, _):
                for kq in range(4):
                    ix = buf[pl.ds(i * 64 + kq * 16, 16)] + off
                    plsc.addupdate_scatter(hist_v, [ix], ones)
                return 0

            lax.fori_loop(0, EPT // 64, body, 0)
            ixt = buf[pl.ds((EPT // 16) * 16, 16)] + off
            plsc.addupdate_scatter(hist_v, [ixt], ones, mask=tail_mask)

        pltpu.sync_copy(hist_v, out_h.at[pl.ds(wid * 9 * N, 9 * N)])

    return k(f_heter, f_h0, f_h1, f_hm0, f_hm1)


def _spmm(z, src2, dst2, ap=NP, cpt=CPT, ncht=NCHT, dep=4):
    @functools.partial(
        pl.kernel,
        out_type=jax.ShapeDtypeStruct((2 * ap, D), jnp.float32),
        mesh=_mesh(),
        compiler_params=pltpu.CompilerParams(needs_layout_passes=False,
                                             use_tc_tiling_on_sc=False),
        scratch_types=[
            pltpu.VMEM_SHARED((ap, D), jnp.float32),
            [pltpu.VMEM((CH, D), jnp.float32)] * dep,
            pltpu.VMEM((cpt, CH), jnp.int32),
            pltpu.VMEM((cpt, CH), jnp.int32),
            [pltpu.SemaphoreType.DMA] * dep,
            [pltpu.SemaphoreType.DMA] * dep,
        ],
    )
    def k(z_h, src_h, dst_h, out_h,
          acc_s, rows, si_big, gi_big, sg, ss):
        rpt = ap // NS
        cid = lax.axis_index("c")
        sid = lax.axis_index("s")
        wid = cid * NS + sid
        row0 = sid * rpt

        z16 = jnp.zeros((16,), jnp.float32)

        def zb(i, _):
            for kq in range(4):
                rows[0][i, pl.ds(kq * 16, 16)] = z16
            return 0

        lax.fori_loop(0, CH, zb, 0)
        nzb = rpt // CH

        def zdma(i, _):
            pltpu.sync_copy(rows[0], acc_s.at[pl.ds(row0 + i * CH, CH)])
            return 0

        lax.fori_loop(0, nzb, zdma, 0)

        r0 = wid * cpt
        pltpu.sync_copy(dst_h.at[pl.ds(r0, cpt)], gi_big)
        pltpu.sync_copy(src_h.at[pl.ds(r0, cpt)], si_big)
        nc = jnp.where(wid == NW - 1, ncht - (NW - 1) * cpt, cpt)

        plsc.subcore_barrier()

        def start_gather(j, b):
            pltpu.async_copy(z_h.at[gi_big.at[j]], rows[b], sg[b])

        def wait_gather(b):
            pltpu.make_async_copy(z_h.at[gi_big.at[0]], rows[b], sg[b]).wait()

        def start_scatter(j, b):
            pltpu.async_copy(rows[b], acc_s.at[si_big.at[j]], ss[b], add=True)

        def wait_scatter(b):
            pltpu.make_async_copy(rows[b], acc_s.at[si_big.at[0]], ss[b]).wait()

        for b in range(dep):
            start_gather(b, b)

        def quad(q, _):
            j0 = dep * q
            for b in range(dep):
                wait_gather(b)
                start_scatter(j0 + b, b)
            for b in range(dep):
                wait_scatter(b)
                start_gather(j0 + dep + b, b)
            return 0

        lax.fori_loop(0, (nc - dep) // dep, quad, 0)

        jl = dep * ((nc - dep) // dep)
        for b in range(dep):
            wait_gather(b)
            start_scatter(jl + b, b)
        for b in range(dep):
            wait_scatter(b)

        def seq(r, _):
            j = jl + dep + r
            pltpu.async_copy(z_h.at[gi_big.at[j]], rows[0], sg[0])
            wait_gather(0)
            pltpu.sync_copy(rows[0], acc_s.at[si_big.at[j]], add=True)
            return 0

        lax.fori_loop(0, nc - jl - dep, seq, 0)

        plsc.subcore_barrier()
        pltpu.sync_copy(acc_s.at[pl.ds(row0, rpt)],
                        out_h.at[pl.ds(cid * ap + row0, rpt)])

    return k(z, src2, dst2)


_R = 1000
_G = N // _R
_RP = 1280
_GP = NP // _RP


def _row_spec(shape_tail):
    return pl.BlockSpec((_R,) + shape_tail, lambda i: (i,) + (0,) * len(shape_tail))


def _rowp_spec(shape_tail):
    return pl.BlockSpec((_RP,) + shape_tail, lambda i: (i,) + (0,) * len(shape_tail))


def _full_spec(shape):
    return pl.BlockSpec(shape, lambda i: (0,) * len(shape))


def _prep(emb, W0, b0, W1, b1, theta):
    def body(e_r, w0_r, b0_r, w1_r, b1_r, th_r, u0_r, c_r, xt_r):
        e = e_r[...]
        u0 = e * jax.nn.sigmoid(jnp.dot(e, w0_r[...], precision=HI) + b0_r[...])
        c = e * jax.nn.sigmoid(jnp.dot(e, w1_r[...], precision=HI) + b1_r[...])
        u0_r[...] = u0
        c_r[...] = c
        xt_r[...] = jnp.dot(c, th_r[...], precision=HI)

    o = jax.ShapeDtypeStruct((NP, D), jnp.float32)
    return pl.pallas_call(
        body,
        grid=(_GP,),
        in_specs=[_rowp_spec((D,)), _full_spec((D, D)), _full_spec((1, D)),
                  _full_spec((D, D)), _full_spec((1, D)), _full_spec((D, D))],
        out_specs=[_rowp_spec((D,))] * 3,
        out_shape=[o, o, o],
    )(emb, W0, b0, W1, b1, theta)


def _factors(per_tile):
    def body(h_r, out_r):
        c = jnp.sum(h_r[...], axis=0)
        deg = c[0:1]
        dsi = lax.rsqrt(jnp.where(deg == 0.0, 1.0, deg))
        inv = jnp.where(c[1:9] > 0.0, 1.0 / jnp.where(c[1:9] > 0.0, c[1:9], 1.0), 0.0)
        out_r[...] = jnp.concatenate([dsi, inv], axis=0)

    return pl.pallas_call(
        body,
        grid=(1,),
        in_specs=[_full_spec((NW, 9, N))],
        out_specs=_full_spec((9, N)),
        out_shape=jax.ShapeDtypeStruct((9, N), jnp.float32),
    )(per_tile)


def _norm_scale(x, dsi):
    def body(x_r, d_r, z_r):
        x = x_r[...]
        nrm = jnp.sqrt(jnp.sum(x * x, axis=1, keepdims=True)) + 1e-12
        z_r[...] = x * d_r[...] / nrm

    return pl.pallas_call(
        body,
        grid=(_GP,),
        in_specs=[_rowp_spec((D,)), _rowp_spec((1,))],
        out_specs=_rowp_spec((D,)),
        out_shape=jax.ShapeDtypeStruct((NP, D), jnp.float32),
    )(x, dsi)


def _combine(p, dsi, S):
    def body(p0_r, p1_r, d_r, s_r, z_r, so_r):
        d = d_r[...]
        x = (p0_r[...] + p1_r[...]) * d
        so_r[...] = s_r[...] + x
        nrm = jnp.sqrt(jnp.sum(x * x, axis=1, keepdims=True)) + 1e-12
        z_r[...] = x * d / nrm

    o = jax.ShapeDtypeStruct((NP, D), jnp.float32)
    return pl.pallas_call(
        body,
        grid=(_GP,),
        in_specs=[_rowp_spec((D,))] * 2 + [_rowp_spec((1,)), _rowp_spec((D,))],
        out_specs=[_rowp_spec((D,))] * 2,
        out_shape=[o, o],
    )(p[:NP], p[NP:], dsi, S)


def _scale2(p0, p1, binv):
    def body(p0_r, p1_r, b_r, m_r):
        m_r[...] = (p0_r[...] + p1_r[...]) * b_r[...]

    return pl.pallas_call(
        body,
        grid=(_GP,),
        in_specs=[_rowp_spec((D,))] * 2 + [_rowp_spec((1,))],
        out_specs=_rowp_spec((D,)),
        out_shape=jax.ShapeDtypeStruct((NP, D), jnp.float32),
    )(p0, p1, binv)


def _gelu(x):
    return 0.5 * x * (1.0 + lax.erf(x * (2.0 ** -0.5)))


def _fusion(q00, q01, dinv0, q10, q11, dinv1, hg_bias, fW1, fb1, fW2, fb2):
    def body(q00_r, q01_r, d0_r, q10_r, q11_r, d1_r, bias_r,
             w1_r, b1_r, w2_r, out_r):
        bias = bias_r[...]
        h0 = (q00_r[...] + q01_r[...]) * d0_r[...] + bias
        h1 = (q10_r[...] + q11_r[...]) * d1_r[...] + bias
        w1t = w1_r[...]
        dn = (((1,), (1,)), ((), ()))
        g0 = _gelu(lax.dot_general(h0, w1t, dn, precision=HI) + b1_r[...])
        g1 = _gelu(lax.dot_general(h1, w1t, dn, precision=HI) + b1_r[...])
        w2 = w2_r[...]
        s0 = lax.dot_general(g0, w2, dn, precision=HI)
        s1 = lax.dot_general(g1, w2, dn, precision=HI)
        mx = jnp.maximum(s0, s1)
        e0 = jnp.exp(s0 - mx)
        e1 = jnp.exp(s1 - mx)
        out_r[...] = (e0 * h0 + e1 * h1) / (e0 + e1)

    return pl.pallas_call(
        body,
        grid=(_G,),
        in_specs=[_row_spec((D,)), _row_spec((D,)), _row_spec((1,)),
                  _row_spec((D,)), _row_spec((D,)), _row_spec((1,)),
                  _full_spec((1, D)),
                  _full_spec((D, D)), _full_spec((1, D)),
                  _full_spec((1, D))],
        out_specs=_row_spec((D,)),
        out_shape=jax.ShapeDtypeStruct((N, D), jnp.float32),
    )(q00, q01, dinv0, q10, q11, dinv1,
      hg_bias.reshape(1, D), fW1, fb1.reshape(1, D), fW2)


def _attention(u0, S, h_emb, hm_emb, att, att_m):
    def body(u0_r, s_r, h_r, hm_r, att_r, am_r, out_r):
        u = 0.1 * u0_r[...] + 0.15 * s_r[...]
        h = h_r[...]
        hm = hm_r[...]
        dn = (((1,), (1,)), ((), ()))
        v = lax.dot_general(am_r[...], att_r[...], dn, precision=HI)
        dn2 = (((1,), (0,)), ((), ()))
        wu = lax.dot_general(u, v, dn2, precision=HI)
        wh = lax.dot_general(h, v, dn2, precision=HI)
        wm = lax.dot_general(hm, v, dn2, precision=HI)
        mx = jnp.maximum(jnp.maximum(wu, wh), wm)
        eu = jnp.exp(wu - mx)
        eh = jnp.exp(wh - mx)
        em = jnp.exp(wm - mx)
        out_r[...] = (eu * u + eh * h + em * hm) / (eu + eh + em)

    return pl.pallas_call(
        body,
        grid=(_G,),
        in_specs=[_row_spec((D,))] * 4 + [_full_spec((1, D)), _full_spec((D, D))],
        out_specs=_row_spec((D,)),
        out_shape=jax.ShapeDtypeStruct((N, D), jnp.float32),
    )(u0, S, h_emb, hm_emb, att, att_m)


def kernel(heter_edge_index, hyper_edge_index_0, hyper_edge_index_1,
           hyper_micro_edge_index_0, hyper_micro_edge_index_1, emb_table,
           W0, b0, W1, b1, theta, hg_bias, att, att_m,
           fus_W1, fus_b1, fus_W2, fus_b2):
    f_heter = heter_edge_index.reshape(-1)
    f_h0 = hyper_edge_index_0.reshape(-1)
    f_h1 = hyper_edge_index_1.reshape(-1)
    f_hm0 = hyper_micro_edge_index_0.reshape(-1)
    f_hm1 = hyper_micro_edge_index_1.reshape(-1)

    hist = _degrees(f_heter, f_h0, f_h1, f_hm0, f_hm1)
    fac = _factors(hist.reshape(NW, 9, N))
    padc = jnp.zeros((NP - N, 1), jnp.float32)

    def _padded(row):
        return jnp.concatenate([row.reshape(N, 1), padc], axis=0)

    dsi = _padded(fac[0])
    dinv = [fac[i].reshape(N, 1) for i in (1, 3, 5, 7)]
    binv = [_padded(fac[i]) for i in (2, 4, 6, 8)]

    emb_p = jnp.concatenate([emb_table, jnp.zeros((NP - N, D), jnp.float32)], axis=0)
    u0, c_emb, xt = _prep(emb_p, W0, b0, W1, b1, theta)
    del c_emb

    def _chunked(ix):
        pad = jnp.zeros((NCHP - NCHT, CH), jnp.int32)
        return jnp.concatenate([ix.reshape(NCHT, CH), pad], axis=0)

    NCHT2 = 2 * E // CH
    CPT2 = 80

    def _chunked2(ix):
        pad2 = jnp.zeros((NW * CPT2 - NCHT2, CH), jnp.int32)
        return jnp.concatenate([ix.reshape(NCHT2, CH), pad2], axis=0)

    def _conv_pair(ei_a, ei_b):
        node = jnp.concatenate([ei_a[0], ei_b[0]])
        he_ofs = jnp.concatenate([ei_a[1], ei_b[1] + NP])
        return _chunked2(he_ofs), _chunked2(node)

    he2m, node2m = _conv_pair(hyper_edge_index_0, hyper_edge_index_1)
    he2u, node2u = _conv_pair(hyper_micro_edge_index_0, hyper_micro_edge_index_1)

    z = _norm_scale(u0, dsi)
    S = jnp.zeros((NP, D), jnp.float32)
    src = _chunked(heter_edge_index[0])
    dst = _chunked(heter_edge_index[1])

    p = _spmm(z, src, dst)
    pm_mac = _spmm(xt, he2m, node2m, ap=2 * NP, cpt=CPT2, ncht=NCHT2, dep=3)
    z, S = _combine(p, dsi, S)
    p = _spmm(z, src, dst)
    pm_mic = _spmm(xt, he2u, node2u, ap=2 * NP, cpt=CPT2, ncht=NCHT2, dep=3)
    z, S = _combine(p, dsi, S)
    m_a = _scale2(pm_mac[0:NP], pm_mac[2 * NP:3 * NP], binv[0])
    m_b = _scale2(pm_mac[NP:2 * NP], pm_mac[3 * NP:4 * NP], binv[1])
    z2m = jnp.concatenate([m_a, m_b], axis=0)
    p = _spmm(z, src, dst)
    po_mac = _spmm(z2m, node2m, he2m, ap=2 * NP, cpt=CPT2, ncht=NCHT2, dep=3)
    z, S = _combine(p, dsi, S)
    m_c = _scale2(pm_mic[0:NP], pm_mic[2 * NP:3 * NP], binv[2])
    m_d = _scale2(pm_mic[NP:2 * NP], pm_mic[3 * NP:4 * NP], binv[3])
    z2u = jnp.concatenate([m_c, m_d], axis=0)
    p = _spmm(z, src, dst)
    po_mic = _spmm(z2u, node2u, he2u, ap=2 * NP, cpt=CPT2, ncht=NCHT2, dep=3)
    z, S = _combine(p, dsi, S)
    p = _spmm(z, src, dst)
    z, S = _combine(p, dsi, S)
    p = _spmm(z, src, dst)
    z, S = _combine(p, dsi, S)
    h_out = [po_mac, po_mic]

    h_emb = _fusion(h_out[0][0:N], h_out[0][2 * NP:2 * NP + N], dinv[0],
                    h_out[0][NP:NP + N], h_out[0][3 * NP:3 * NP + N], dinv[1],
                    hg_bias, fus_W1, fus_b1, fus_W2, fus_b2)
    h_emb_micro = _fusion(h_out[1][0:N], h_out[1][2 * NP:2 * NP + N], dinv[2],
                          h_out[1][NP:NP + N], h_out[1][3 * NP:3 * NP + N], dinv[3],
                          hg_bias, fus_W1, fus_b1, fus_W2, fus_b2)

    return _attention(u0[:N], S[:N], h_emb, h_emb_micro, att, att_m)
